# baseline (speedup 1.0000x reference)
import jax
import jax.numpy as jnp
from jax import lax
from jax.experimental import pallas as pl
from jax.experimental.pallas import tpu as pltpu
from jax.experimental.pallas import tpu_sc as plsc

M = 262144
D = 64
DP = 128
B = 131072
H = 128

NC = 2
NS = 16
L = 16

MH = M // NC
CHUNK = 8192
NPASS = MH // CHUNK
SCAN = B // NS
ROWS_PT = CHUNK // NS
ZROWS = 32
CH = 64
NTRASH = 16
CAP = SCAN + 2 * CH


def _mlp_body(valT_ref, w1_ref, b1_ref, w2_ref, b2_ref, out_ref):
    h = lax.dot_general(valT_ref[...], w1_ref[...],
                        (((0,), (0,)), ((), ())),
                        preferred_element_type=jnp.float32)
    h = h + b1_ref[...]
    h = jnp.where(h >= 0, h, 0.25 * h)
    u = jnp.dot(h, w2_ref[...], preferred_element_type=jnp.float32)
    u = u + b2_ref[...]
    out_ref[...] = jnp.where(u >= 0, u, 0.25 * u)


def _mlp(valT, W1, b1, W2p, b2p):
    blk = 8192
    grid = (B // blk,)
    return pl.pallas_call(
        _mlp_body,
        grid=grid,
        in_specs=[
            pl.BlockSpec((D, blk), lambda i: (0, i)),
            pl.BlockSpec((D, H), lambda i: (0, 0)),
            pl.BlockSpec((1, H), lambda i: (0, 0)),
            pl.BlockSpec((H, DP), lambda i: (0, 0)),
            pl.BlockSpec((1, DP), lambda i: (0, 0)),
        ],
        out_specs=pl.BlockSpec((blk, DP), lambda i: (i, 0)),
        out_shape=jax.ShapeDtypeStruct((B, DP), jnp.float32),
    )(valT, W1, b1.reshape(1, H), W2p, b2p)


UNR = 4


HS = None


def _sc_body(idx_hbm, upd_hbm, delta_hbm,
             idx_v, myidx, mypos, posf, rowf, pos2a, row2a, pos2b, row2b,
             gbufa, gbufb, zbuf, shared, sema, semb, zsem):
    c = lax.axis_index("c")
    s = lax.axis_index("s")

    pltpu.sync_copy(idx_hbm.at[pl.ds(s * SCAN, SCAN)], idx_v)

    lanes = lax.iota(jnp.int32, L)
    zeros = jnp.zeros((L,), jnp.float32)
    for r in range(ZROWS):
        for g in range(DP // L):
            zbuf[r, pl.ds(g * L, L)] = zeros

    vbase = c * MH

    def pre_step(i, n):
        pfs, vs, ms = [], [], []
        for t in range(UNR):
            v = idx_v[pl.ds((i * UNR + t) * L, L)]
            m = (v >= vbase) & (v < vbase + MH)
            pfs.append(plsc.cumsum(m.astype(jnp.int32)))
            vs.append(v)
            ms.append(m)
        for t in range(UNR):
            off = n + pfs[t] - 1
            pos = s * SCAN + (i * UNR + t) * L + lanes
            plsc.store_scatter(mypos, [off], pos, mask=ms[t])
            plsc.store_scatter(myidx, [off], vs[t], mask=ms[t])
            n = n + pfs[t][L - 1]
        return n

    nmy = lax.fori_loop(0, SCAN // (L * UNR), pre_step, jnp.int32(0))

    p0 = (nmy // L) * L
    for t in range(UNR + 1):
        a = p0 + t * L
        keep = (a + lanes) < nmy
        iv = myidx[pl.ds(a, L)]
        myidx[pl.ds(a, L)] = jnp.where(keep, iv, jnp.int32(-1))
    nblk = (nmy + L * UNR - 1) // (L * UNR)

    def fire_zero(base, rows):
        for t in range(rows // ZROWS):
            pltpu.async_copy(
                zbuf, shared.at[pl.ds(base + t * ZROWS, ZROWS)], zsem)

    def drain_zero(rows):
        for t in range(rows // ZROWS):
            pltpu.make_async_copy(
                zbuf, shared.at[pl.ds(s * ROWS_PT + t * ZROWS, ZROWS)],
                zsem).wait()

    fire_zero(s * ROWS_PT, ROWS_PT)
    drain_zero(ROWS_PT)

    def one_pass(k, _):
        gbase = vbase + k * CHUNK

        def scan_step(i, n):
            pfs, vs, ms, ps = [], [], [], []
            for t in range(UNR):
                v = myidx[pl.ds((i * UNR + t) * L, L)]
                p = mypos[pl.ds((i * UNR + t) * L, L)]
                lv = v - gbase
                m = (lv >= 0) & (lv < CHUNK)
                pfs.append(plsc.cumsum(m.astype(jnp.int32)))
                vs.append(lv)
                ms.append(m)
                ps.append(p)
            for t in range(UNR):
                off = n + pfs[t] - 1
                plsc.store_scatter(posf, [off], ps[t], mask=ms[t])
                plsc.store_scatter(rowf, [off], vs[t], mask=ms[t])
                n = n + pfs[t][L - 1]
            return n

        n = lax.fori_loop(0, nblk, scan_step, jnp.int32(0))

        a0 = (n // L) * L
        for t in range(CH // L + 1):
            a = a0 + t * L
            keep = (a + lanes) < n
            pv = posf[pl.ds(a, L)]
            rv = rowf[pl.ds(a, L)]
            posf[pl.ds(a, L)] = jnp.where(keep, pv, lanes)
            rowf[pl.ds(a, L)] = jnp.where(keep, rv, CHUNK + lanes)

        plsc.subcore_barrier()

        nch = jnp.maximum((n + CH - 1) // CH, 1)

        def fill(j, posb, rowb):
            for t in range(CH // L):
                posb[pl.ds(t * L, L)] = posf[pl.ds(j * CH + t * L, L)]
                rowb[pl.ds(t * L, L)] = rowf[pl.ds(j * CH + t * L, L)]

        fill(0, pos2a, row2a)
        pltpu.async_copy(upd_hbm.at[pos2a], gbufa, sema)

        def pair_step(h, _):
            j0 = h * 2
            @pl.when(j0 + 1 < nch)
            def _():
                fill(j0 + 1, pos2b, row2b)
                pltpu.async_copy(upd_hbm.at[pos2b], gbufb, semb)
            pltpu.make_async_copy(upd_hbm.at[pos2a], gbufa, sema).wait()
            pltpu.sync_copy(gbufa, shared.at[row2a], add=True)
            @pl.when(j0 + 2 < nch)
            def _():
                fill(j0 + 2, pos2a, row2a)
                pltpu.async_copy(upd_hbm.at[pos2a], gbufa, sema)
            @pl.when(j0 + 1 < nch)
            def _():
                pltpu.make_async_copy(upd_hbm.at[pos2b], gbufb, semb).wait()
                pltpu.sync_copy(gbufb, shared.at[row2b], add=True)
            return 0

        lax.fori_loop(0, (nch + 1) // 2, pair_step, 0)
        plsc.subcore_barrier()

        pltpu.sync_copy(shared.at[pl.ds(s * ROWS_PT, ROWS_PT)],
                        delta_hbm.at[pl.ds(gbase + s * ROWS_PT, ROWS_PT)])
        return 0

    lax.fori_loop(0, NPASS, one_pass, 0)


def _scatter(idx, upd):
    mesh = plsc.VectorSubcoreMesh(core_axis_name="c", subcore_axis_name="s")
    f = pl.kernel(
        _sc_body,
        out_type=jax.ShapeDtypeStruct((M, DP), jnp.float32),
        mesh=mesh,
        compiler_params=pltpu.CompilerParams(needs_layout_passes=False),
        scratch_types=[
            pltpu.VMEM((SCAN,), jnp.int32),
            pltpu.VMEM((CAP,), jnp.int32),
            pltpu.VMEM((CAP,), jnp.int32),
            pltpu.VMEM((CAP,), jnp.int32),
            pltpu.VMEM((CAP,), jnp.int32),
            pltpu.VMEM((CH,), jnp.int32),
            pltpu.VMEM((CH,), jnp.int32),
            pltpu.VMEM((CH,), jnp.int32),
            pltpu.VMEM((CH,), jnp.int32),
            pltpu.VMEM((CH, DP), jnp.float32),
            pltpu.VMEM((CH, DP), jnp.float32),
            pltpu.VMEM((ZROWS, DP), jnp.float32),
            pltpu.VMEM_SHARED((CHUNK + NTRASH, DP), jnp.float32),
            pltpu.SemaphoreType.DMA,
            pltpu.SemaphoreType.DMA,
            pltpu.SemaphoreType.DMA,
        ],
    )
    return f(idx, upd)


def _merge_body(memT_ref, delta_ref, prev_ref, outT_ref):
    i = pl.program_id(0)
    w = jnp.where((i == 0) | (i == M // 2 // CHUNK), 0.0, 1.0)
    d = delta_ref[:, :D] - w * prev_ref[:, :D]
    outT_ref[...] = memT_ref[...] + d.T


def _merge(memT, delta):
    blk = CHUNK
    grid = (M // blk,)
    return pl.pallas_call(
        _merge_body,
        grid=grid,
        in_specs=[
            pl.BlockSpec((D, blk), lambda i: (0, i)),
            pl.BlockSpec((blk, DP), lambda i: (i, 0)),
            pl.BlockSpec((blk, DP), lambda i: (jnp.maximum(i - 1, 0), 0)),
        ],
        out_specs=pl.BlockSpec((D, blk), lambda i: (0, i)),
        out_shape=jax.ShapeDtypeStruct((D, M), jnp.float32),
    )(memT, delta, delta)


def kernel(mem, idx, val, W1, b1, W2, b2):
    W2p = jnp.zeros((H, DP), jnp.float32).at[:, :D].set(W2)
    b2p = jnp.zeros((1, DP), jnp.float32).at[:, :D].set(b2)
    upd = _mlp(val.T, W1, b1, W2p, b2p)
    delta = _scatter(idx, upd)
    return _merge(mem.T, delta).T


# speedup vs baseline: 1.0875x; 1.0875x over previous
import jax
import jax.numpy as jnp
from jax import lax
from jax.experimental import pallas as pl
from jax.experimental.pallas import tpu as pltpu
from jax.experimental.pallas import tpu_sc as plsc

M = 262144
D = 64
DP = 128
B = 131072
H = 128

NC = 2
NS = 16
L = 16

MH = M // NC
CHUNK = 8192
NPASS = MH // CHUNK
SCAN = B // NS
ROWS_PT = CHUNK // NS
ZROWS = 32
CH = 64
NTRASH = 16
CAP = SCAN + 2 * CH


def _mlp_body(valT_ref, w1_ref, b1_ref, w2_ref, b2_ref, out_ref):
    h = lax.dot_general(valT_ref[...], w1_ref[...],
                        (((0,), (0,)), ((), ())),
                        preferred_element_type=jnp.float32)
    h = h + b1_ref[...]
    h = jnp.where(h >= 0, h, 0.25 * h)
    u = jnp.dot(h, w2_ref[...], preferred_element_type=jnp.float32)
    u = u + b2_ref[...]
    out_ref[...] = jnp.where(u >= 0, u, 0.25 * u)


def _mlp(valT, W1, b1, W2p, b2p):
    blk = 8192
    grid = (B // blk,)
    return pl.pallas_call(
        _mlp_body,
        grid=grid,
        in_specs=[
            pl.BlockSpec((D, blk), lambda i: (0, i)),
            pl.BlockSpec((D, H), lambda i: (0, 0)),
            pl.BlockSpec((1, H), lambda i: (0, 0)),
            pl.BlockSpec((H, DP), lambda i: (0, 0)),
            pl.BlockSpec((1, DP), lambda i: (0, 0)),
        ],
        out_specs=pl.BlockSpec((blk, DP), lambda i: (i, 0)),
        out_shape=jax.ShapeDtypeStruct((B, DP), jnp.float32),
    )(valT, W1, b1.reshape(1, H), W2p, b2p)


UNR = 4


HS = None


def _sc_body(idx_hbm, upd_hbm, delta_hbm,
             idx_v, myidx, mypos, posf, rowf, pos2a, row2a, pos2b, row2b,
             gbufa, gbufb, zbuf, shared, sema, semb, zsem, wsema, wsemb):
    c = lax.axis_index("c")
    s = lax.axis_index("s")

    pltpu.sync_copy(idx_hbm.at[pl.ds(s * SCAN, SCAN)], idx_v)

    lanes = lax.iota(jnp.int32, L)
    zeros = jnp.zeros((L,), jnp.float32)
    for r in range(ZROWS):
        for g in range(DP // L):
            zbuf[r, pl.ds(g * L, L)] = zeros

    vbase = c * MH

    def pre_step(i, n):
        pfs, vs, ms = [], [], []
        for t in range(UNR):
            v = idx_v[pl.ds((i * UNR + t) * L, L)]
            m = (v >= vbase) & (v < vbase + MH)
            pfs.append(plsc.cumsum(m.astype(jnp.int32)))
            vs.append(v)
            ms.append(m)
        for t in range(UNR):
            off = n + pfs[t] - 1
            pos = s * SCAN + (i * UNR + t) * L + lanes
            plsc.store_scatter(mypos, [off], pos, mask=ms[t])
            plsc.store_scatter(myidx, [off], vs[t], mask=ms[t])
            n = n + pfs[t][L - 1]
        return n

    nmy = lax.fori_loop(0, SCAN // (L * UNR), pre_step, jnp.int32(0))

    p0 = (nmy // L) * L
    for t in range(UNR + 1):
        a = p0 + t * L
        keep = (a + lanes) < nmy
        iv = myidx[pl.ds(a, L)]
        myidx[pl.ds(a, L)] = jnp.where(keep, iv, jnp.int32(-1))
    nblk = (nmy + L * UNR - 1) // (L * UNR)

    def fire_zero(base, rows):
        for t in range(rows // ZROWS):
            pltpu.async_copy(
                zbuf, shared.at[pl.ds(base + t * ZROWS, ZROWS)], zsem)

    def drain_zero(rows):
        for t in range(rows // ZROWS):
            pltpu.make_async_copy(
                zbuf, shared.at[pl.ds(s * ROWS_PT + t * ZROWS, ZROWS)],
                zsem).wait()

    fire_zero(s * ROWS_PT, ROWS_PT)

    def one_pass(k, _):
        gbase = vbase + k * CHUNK

        def scan_step(i, n):
            pfs, vs, ms, ps = [], [], [], []
            for t in range(UNR):
                v = myidx[pl.ds((i * UNR + t) * L, L)]
                p = mypos[pl.ds((i * UNR + t) * L, L)]
                lv = v - gbase
                m = (lv >= 0) & (lv < CHUNK)
                pfs.append(plsc.cumsum(m.astype(jnp.int32)))
                vs.append(lv)
                ms.append(m)
                ps.append(p)
            for t in range(UNR):
                off = n + pfs[t] - 1
                plsc.store_scatter(posf, [off], ps[t], mask=ms[t])
                plsc.store_scatter(rowf, [off], vs[t], mask=ms[t])
                n = n + pfs[t][L - 1]
            return n

        n = lax.fori_loop(0, nblk, scan_step, jnp.int32(0))

        a0 = (n // L) * L
        for t in range(CH // L + 1):
            a = a0 + t * L
            keep = (a + lanes) < n
            pv = posf[pl.ds(a, L)]
            rv = rowf[pl.ds(a, L)]
            posf[pl.ds(a, L)] = jnp.where(keep, pv, lanes)
            rowf[pl.ds(a, L)] = jnp.where(keep, rv, CHUNK + lanes)

        drain_zero(ROWS_PT)
        plsc.subcore_barrier()

        nch = jnp.maximum((n + CH - 1) // CH, 1)

        def fill(j, posb, rowb):
            for t in range(CH // L):
                posb[pl.ds(t * L, L)] = posf[pl.ds(j * CH + t * L, L)]
                rowb[pl.ds(t * L, L)] = rowf[pl.ds(j * CH + t * L, L)]

        fill(0, pos2a, row2a)
        pltpu.async_copy(upd_hbm.at[pos2a], gbufa, sema)

        def pair_step(h, _):
            j0 = h * 2
            @pl.when(j0 + 1 < nch)
            def _():
                fill(j0 + 1, pos2b, row2b)
                pltpu.async_copy(upd_hbm.at[pos2b], gbufb, semb)
            pltpu.make_async_copy(upd_hbm.at[pos2a], gbufa, sema).wait()
            pltpu.sync_copy(gbufa, shared.at[row2a], add=True)
            @pl.when(j0 + 2 < nch)
            def _():
                fill(j0 + 2, pos2a, row2a)
                pltpu.async_copy(upd_hbm.at[pos2a], gbufa, sema)
            @pl.when(j0 + 1 < nch)
            def _():
                pltpu.make_async_copy(upd_hbm.at[pos2b], gbufb, semb).wait()
                pltpu.sync_copy(gbufb, shared.at[row2b], add=True)
            return 0

        lax.fori_loop(0, (nch + 1) // 2, pair_step, 0)
        plsc.subcore_barrier()

        half = ROWS_PT // 2
        wa = pltpu.async_copy(
            shared.at[pl.ds(s * ROWS_PT, half)],
            delta_hbm.at[pl.ds(gbase + s * ROWS_PT, half)], wsema)
        wb = pltpu.async_copy(
            shared.at[pl.ds(s * ROWS_PT + half, half)],
            delta_hbm.at[pl.ds(gbase + s * ROWS_PT + half, half)], wsemb)
        wa.wait()
        fire_zero(s * ROWS_PT, half)
        wb.wait()
        fire_zero(s * ROWS_PT + half, half)
        return 0

    lax.fori_loop(0, NPASS, one_pass, 0)
    drain_zero(ROWS_PT)


def _scatter(idx, upd):
    mesh = plsc.VectorSubcoreMesh(core_axis_name="c", subcore_axis_name="s")
    f = pl.kernel(
        _sc_body,
        out_type=jax.ShapeDtypeStruct((M, DP), jnp.float32),
        mesh=mesh,
        compiler_params=pltpu.CompilerParams(needs_layout_passes=False),
        scratch_types=[
            pltpu.VMEM((SCAN,), jnp.int32),
            pltpu.VMEM((CAP,), jnp.int32),
            pltpu.VMEM((CAP,), jnp.int32),
            pltpu.VMEM((CAP,), jnp.int32),
            pltpu.VMEM((CAP,), jnp.int32),
            pltpu.VMEM((CH,), jnp.int32),
            pltpu.VMEM((CH,), jnp.int32),
            pltpu.VMEM((CH,), jnp.int32),
            pltpu.VMEM((CH,), jnp.int32),
            pltpu.VMEM((CH, DP), jnp.float32),
            pltpu.VMEM((CH, DP), jnp.float32),
            pltpu.VMEM((ZROWS, DP), jnp.float32),
            pltpu.VMEM_SHARED((CHUNK + NTRASH, DP), jnp.float32),
            pltpu.SemaphoreType.DMA,
            pltpu.SemaphoreType.DMA,
            pltpu.SemaphoreType.DMA,
            pltpu.SemaphoreType.DMA,
            pltpu.SemaphoreType.DMA,
        ],
    )
    return f(idx, upd)


def _merge_body(memT_ref, delta_ref, outT_ref):
    outT_ref[...] = memT_ref[...] + delta_ref[:, :D].T


def _merge(memT, delta):
    blk = 8192
    grid = (M // blk,)
    return pl.pallas_call(
        _merge_body,
        grid=grid,
        in_specs=[
            pl.BlockSpec((D, blk), lambda i: (0, i)),
            pl.BlockSpec((blk, DP), lambda i: (i, 0)),
        ],
        out_specs=pl.BlockSpec((D, blk), lambda i: (0, i)),
        out_shape=jax.ShapeDtypeStruct((D, M), jnp.float32),
    )(memT, delta)


def kernel(mem, idx, val, W1, b1, W2, b2):
    W2p = jnp.zeros((H, DP), jnp.float32).at[:, :D].set(W2)
    b2p = jnp.zeros((1, DP), jnp.float32).at[:, :D].set(b2)
    upd = _mlp(val.T, W1, b1, W2p, b2p)
    delta = _scatter(idx, upd)
    return _merge(mem.T, delta).T


# speedup vs baseline: 1.0910x; 1.0031x over previous
import jax
import jax.numpy as jnp
from jax import lax
from jax.experimental import pallas as pl
from jax.experimental.pallas import tpu as pltpu
from jax.experimental.pallas import tpu_sc as plsc

M = 262144
D = 64
DP = 128
B = 131072
H = 128

NC = 2
NS = 16
L = 16

MH = M // NC
CHUNK = 8192
NPASS = MH // CHUNK
SCAN = B // NS
ROWS_PT = CHUNK // NS
ZROWS = 32
CH = 64
NTRASH = 16
CAP = SCAN + 2 * CH


def _mlp_body(valT_ref, w1_ref, b1_ref, w2_ref, b2_ref, out_ref):
    h = lax.dot_general(valT_ref[...], w1_ref[...],
                        (((0,), (0,)), ((), ())),
                        preferred_element_type=jnp.float32)
    h = h + b1_ref[...]
    h = jnp.where(h >= 0, h, 0.25 * h)
    u = jnp.dot(h, w2_ref[...], preferred_element_type=jnp.float32)
    u = u + b2_ref[...]
    out_ref[...] = jnp.where(u >= 0, u, 0.25 * u)


def _mlp(valT, W1, b1, W2p, b2p):
    blk = 8192
    grid = (B // blk,)
    return pl.pallas_call(
        _mlp_body,
        grid=grid,
        in_specs=[
            pl.BlockSpec((D, blk), lambda i: (0, i)),
            pl.BlockSpec((D, H), lambda i: (0, 0)),
            pl.BlockSpec((1, H), lambda i: (0, 0)),
            pl.BlockSpec((H, DP), lambda i: (0, 0)),
            pl.BlockSpec((1, DP), lambda i: (0, 0)),
        ],
        out_specs=pl.BlockSpec((blk, DP), lambda i: (i, 0)),
        out_shape=jax.ShapeDtypeStruct((B, DP), jnp.float32),
    )(valT, W1, b1.reshape(1, H), W2p, b2p)


UNR = 8


HS = None


def _sc_body(idx_hbm, upd_hbm, delta_hbm,
             idx_v, myidx, mypos, posf, rowf, pos2a, row2a, pos2b, row2b,
             gbufa, gbufb, zbuf, shared, sema, semb, zsem, wsema, wsemb):
    c = lax.axis_index("c")
    s = lax.axis_index("s")

    pltpu.sync_copy(idx_hbm.at[pl.ds(s * SCAN, SCAN)], idx_v)

    lanes = lax.iota(jnp.int32, L)
    zeros = jnp.zeros((L,), jnp.float32)
    for r in range(ZROWS):
        for g in range(DP // L):
            zbuf[r, pl.ds(g * L, L)] = zeros

    vbase = c * MH

    def pre_step(i, n):
        pfs, vs, ms = [], [], []
        for t in range(UNR):
            v = idx_v[pl.ds((i * UNR + t) * L, L)]
            m = (v >= vbase) & (v < vbase + MH)
            pfs.append(plsc.cumsum(m.astype(jnp.int32)))
            vs.append(v)
            ms.append(m)
        for t in range(UNR):
            off = n + pfs[t] - 1
            pos = s * SCAN + (i * UNR + t) * L + lanes
            plsc.store_scatter(mypos, [off], pos, mask=ms[t])
            plsc.store_scatter(myidx, [off], vs[t], mask=ms[t])
            n = n + pfs[t][L - 1]
        return n

    nmy = lax.fori_loop(0, SCAN // (L * UNR), pre_step, jnp.int32(0))

    p0 = (nmy // L) * L
    for t in range(UNR + 1):
        a = p0 + t * L
        keep = (a + lanes) < nmy
        iv = myidx[pl.ds(a, L)]
        myidx[pl.ds(a, L)] = jnp.where(keep, iv, jnp.int32(-1))
    nblk = (nmy + L * UNR - 1) // (L * UNR)

    def fire_zero(base, rows):
        for t in range(rows // ZROWS):
            pltpu.async_copy(
                zbuf, shared.at[pl.ds(base + t * ZROWS, ZROWS)], zsem)

    def drain_zero(rows):
        for t in range(rows // ZROWS):
            pltpu.make_async_copy(
                zbuf, shared.at[pl.ds(s * ROWS_PT + t * ZROWS, ZROWS)],
                zsem).wait()

    fire_zero(s * ROWS_PT, ROWS_PT)

    def one_pass(k, _):
        gbase = vbase + k * CHUNK

        def scan_step(i, n):
            pfs, vs, ms, ps = [], [], [], []
            for t in range(UNR):
                v = myidx[pl.ds((i * UNR + t) * L, L)]
                p = mypos[pl.ds((i * UNR + t) * L, L)]
                lv = v - gbase
                m = (lv >= 0) & (lv < CHUNK)
                pfs.append(plsc.cumsum(m.astype(jnp.int32)))
                vs.append(lv)
                ms.append(m)
                ps.append(p)
            for t in range(UNR):
                off = n + pfs[t] - 1
                plsc.store_scatter(posf, [off], ps[t], mask=ms[t])
                plsc.store_scatter(rowf, [off], vs[t], mask=ms[t])
                n = n + pfs[t][L - 1]
            return n

        n = lax.fori_loop(0, nblk, scan_step, jnp.int32(0))

        a0 = (n // L) * L
        for t in range(CH // L + 1):
            a = a0 + t * L
            keep = (a + lanes) < n
            pv = posf[pl.ds(a, L)]
            rv = rowf[pl.ds(a, L)]
            posf[pl.ds(a, L)] = jnp.where(keep, pv, lanes)
            rowf[pl.ds(a, L)] = jnp.where(keep, rv, CHUNK + lanes)

        drain_zero(ROWS_PT)
        plsc.subcore_barrier()

        nch = jnp.maximum((n + CH - 1) // CH, 1)

        def fill(j, posb, rowb):
            for t in range(CH // L):
                posb[pl.ds(t * L, L)] = posf[pl.ds(j * CH + t * L, L)]
                rowb[pl.ds(t * L, L)] = rowf[pl.ds(j * CH + t * L, L)]

        fill(0, pos2a, row2a)
        pltpu.async_copy(upd_hbm.at[pos2a], gbufa, sema)

        def pair_step(h, _):
            j0 = h * 2
            @pl.when(j0 + 1 < nch)
            def _():
                fill(j0 + 1, pos2b, row2b)
                pltpu.async_copy(upd_hbm.at[pos2b], gbufb, semb)
            pltpu.make_async_copy(upd_hbm.at[pos2a], gbufa, sema).wait()
            pltpu.sync_copy(gbufa, shared.at[row2a], add=True)
            @pl.when(j0 + 2 < nch)
            def _():
                fill(j0 + 2, pos2a, row2a)
                pltpu.async_copy(upd_hbm.at[pos2a], gbufa, sema)
            @pl.when(j0 + 1 < nch)
            def _():
                pltpu.make_async_copy(upd_hbm.at[pos2b], gbufb, semb).wait()
                pltpu.sync_copy(gbufb, shared.at[row2b], add=True)
            return 0

        lax.fori_loop(0, (nch + 1) // 2, pair_step, 0)
        plsc.subcore_barrier()

        half = ROWS_PT // 2
        wa = pltpu.async_copy(
            shared.at[pl.ds(s * ROWS_PT, half)],
            delta_hbm.at[pl.ds(gbase + s * ROWS_PT, half)], wsema)
        wb = pltpu.async_copy(
            shared.at[pl.ds(s * ROWS_PT + half, half)],
            delta_hbm.at[pl.ds(gbase + s * ROWS_PT + half, half)], wsemb)
        wa.wait()
        fire_zero(s * ROWS_PT, half)
        wb.wait()
        fire_zero(s * ROWS_PT + half, half)
        return 0

    lax.fori_loop(0, NPASS, one_pass, 0)
    drain_zero(ROWS_PT)


def _scatter(idx, upd):
    mesh = plsc.VectorSubcoreMesh(core_axis_name="c", subcore_axis_name="s")
    f = pl.kernel(
        _sc_body,
        out_type=jax.ShapeDtypeStruct((M, DP), jnp.float32),
        mesh=mesh,
        compiler_params=pltpu.CompilerParams(needs_layout_passes=False),
        scratch_types=[
            pltpu.VMEM((SCAN,), jnp.int32),
            pltpu.VMEM((CAP,), jnp.int32),
            pltpu.VMEM((CAP,), jnp.int32),
            pltpu.VMEM((CAP,), jnp.int32),
            pltpu.VMEM((CAP,), jnp.int32),
            pltpu.VMEM((CH,), jnp.int32),
            pltpu.VMEM((CH,), jnp.int32),
            pltpu.VMEM((CH,), jnp.int32),
            pltpu.VMEM((CH,), jnp.int32),
            pltpu.VMEM((CH, DP), jnp.float32),
            pltpu.VMEM((CH, DP), jnp.float32),
            pltpu.VMEM((ZROWS, DP), jnp.float32),
            pltpu.VMEM_SHARED((CHUNK + NTRASH, DP), jnp.float32),
            pltpu.SemaphoreType.DMA,
            pltpu.SemaphoreType.DMA,
            pltpu.SemaphoreType.DMA,
            pltpu.SemaphoreType.DMA,
            pltpu.SemaphoreType.DMA,
        ],
    )
    return f(idx, upd)


def _merge_body(memT_ref, delta_ref, outT_ref):
    outT_ref[...] = memT_ref[...] + delta_ref[:, :D].T


def _merge(memT, delta):
    blk = 8192
    grid = (M // blk,)
    return pl.pallas_call(
        _merge_body,
        grid=grid,
        in_specs=[
            pl.BlockSpec((D, blk), lambda i: (0, i)),
            pl.BlockSpec((blk, DP), lambda i: (i, 0)),
        ],
        out_specs=pl.BlockSpec((D, blk), lambda i: (0, i)),
        out_shape=jax.ShapeDtypeStruct((D, M), jnp.float32),
    )(memT, delta)


def kernel(mem, idx, val, W1, b1, W2, b2):
    W2p = jnp.zeros((H, DP), jnp.float32).at[:, :D].set(W2)
    b2p = jnp.zeros((1, DP), jnp.float32).at[:, :D].set(b2)
    upd = _mlp(val.T, W1, b1, W2p, b2p)
    delta = _scatter(idx, upd)
    return _merge(mem.T, delta).T
